# baseline (speedup 1.0000x reference)
import functools

import jax
import jax.numpy as jnp
from jax import lax
from jax.experimental import pallas as pl
from jax.experimental.pallas import tpu as pltpu
from jax.experimental.pallas import tpu_sc as plsc

_N = 10000
_E = 160000
_IN_FEAT = 32
_D = 256
_OUT_DIM = 256
_HOPS = 3
_NTYPES = 10

_NP = 10240
_EP = 163840
_HALF = _NP // 2
_NSUB = 16
_CH = 128
_PER_SUB = _EP // _NSUB
_NCHUNK = _PER_SUB // _CH
_ROWS_PER_SUB = _HALF // _NSUB
_OCH = 64
_NOCH = _ROWS_PER_SUB // _OCH

_DH2 = _D // 2
_DW = 128
_BR = 1280
_NBLK = _NP // _BR


def _emit_ab(h, w1_ref, w2_ref, mb_ref, alo_ref, ahi_ref, blo_ref, bhi_ref):
    a = jnp.dot(h, w1_ref[...], preferred_element_type=jnp.float32)
    b = jnp.dot(h, w2_ref[...], preferred_element_type=jnp.float32) + mb_ref[...]
    alo_ref[...] = a[:, :_DH2]
    ahi_ref[...] = a[:, _DH2:]
    blo_ref[...] = b[:, :_DH2]
    bhi_ref[...] = b[:, _DH2:]


def _tc_pre_body(nodes_ref, oh_ref, pw_ref, pb_ref, te_ref, w1_ref, w2_ref,
                 mb_ref, h_ref, alo_ref, ahi_ref, blo_ref, bhi_ref):
    h = jnp.dot(nodes_ref[...], pw_ref[...],
                preferred_element_type=jnp.float32) + pb_ref[...]
    h = h + jnp.dot(oh_ref[...], te_ref[...],
                    preferred_element_type=jnp.float32)
    h_ref[...] = h
    _emit_ab(h, w1_ref, w2_ref, mb_ref, alo_ref, ahi_ref, blo_ref, bhi_ref)


def _update(h_ref, agglo_ref, agghi_ref, blo_ref, bhi_ref, deg_ref,
            u1_ref, u2lo_ref, u2hi_ref, ub_ref):
    deg = deg_ref[:, 0:1]
    lo = agglo_ref[...] + deg * blo_ref[...]
    hi = agghi_ref[...] + deg * bhi_ref[...]
    hn = (jnp.dot(h_ref[...], u1_ref[...], preferred_element_type=jnp.float32)
          + jnp.dot(lo, u2lo_ref[...], preferred_element_type=jnp.float32)
          + jnp.dot(hi, u2hi_ref[...], preferred_element_type=jnp.float32)
          + ub_ref[...])
    return jnp.maximum(hn, 0.0)


def _tc_mid_body(h_ref, agglo_ref, agghi_ref, blo_ref, bhi_ref, deg_ref,
                 u1_ref, u2lo_ref, u2hi_ref,
                 ub_ref, w1_ref, w2_ref, mb_ref, hn_ref,
                 alo_ref, ahi_ref, nblo_ref, nbhi_ref):
    hn = _update(h_ref, agglo_ref, agghi_ref, blo_ref, bhi_ref, deg_ref,
                 u1_ref, u2lo_ref, u2hi_ref, ub_ref)
    hn_ref[...] = hn
    _emit_ab(hn, w1_ref, w2_ref, mb_ref, alo_ref, ahi_ref, nblo_ref, nbhi_ref)


def _tc_fin_body(h_ref, agglo_ref, agghi_ref, blo_ref, bhi_ref, deg_ref,
                 u1_ref, u2lo_ref, u2hi_ref,
                 ub_ref, ow_ref, ob_ref, out_ref, acc_ref):
    i = pl.program_id(0)
    hn = _update(h_ref, agglo_ref, agghi_ref, blo_ref, bhi_ref, deg_ref,
                 u1_ref, u2lo_ref, u2hi_ref, ub_ref)
    row = i * _BR + lax.broadcasted_iota(jnp.int32, (_BR, _D), 0)
    hm = jnp.where(row < _N, hn, -1e30)
    m = jnp.max(hm, axis=0, keepdims=True)

    @pl.when(i == 0)
    def _():
        acc_ref[...] = jnp.full((8, _D), -1e30, jnp.float32)

    acc_ref[0:1, :] = jnp.maximum(acc_ref[0:1, :], m)

    @pl.when(i == _NBLK - 1)
    def _():
        g = acc_ref[0:1, :]
        out_ref[...] = jnp.dot(g, ow_ref[...],
                               preferred_element_type=jnp.float32) + ob_ref[...]


def _row_spec(last):
    return pl.BlockSpec((_BR, last), lambda i: (i, 0))


def _full_spec(shape):
    return pl.BlockSpec(shape, lambda i: (0, 0))


_tc_pre = pl.pallas_call(
    _tc_pre_body,
    grid=(_NBLK,),
    in_specs=[
        _row_spec(_IN_FEAT), _row_spec(16),
        _full_spec((_IN_FEAT, _D)), _full_spec((1, _D)),
        _full_spec((16, _D)), _full_spec((_D, _D)), _full_spec((_D, _D)),
        _full_spec((1, _D)),
    ],
    out_specs=[_row_spec(_D)] + [_row_spec(_DH2)] * 4,
    out_shape=([jax.ShapeDtypeStruct((_NP, _D), jnp.float32)]
               + [jax.ShapeDtypeStruct((_NP, _DH2), jnp.float32)] * 4),
)

_tc_mid = pl.pallas_call(
    _tc_mid_body,
    grid=(_NBLK,),
    in_specs=[
        _row_spec(_D), _row_spec(_DH2), _row_spec(_DH2),
        _row_spec(_DH2), _row_spec(_DH2), _row_spec(128),
        _full_spec((_D, _D)), _full_spec((_DH2, _D)), _full_spec((_DH2, _D)),
        _full_spec((1, _D)),
        _full_spec((_D, _D)), _full_spec((_D, _D)), _full_spec((1, _D)),
    ],
    out_specs=[_row_spec(_D)] + [_row_spec(_DH2)] * 4,
    out_shape=([jax.ShapeDtypeStruct((_NP, _D), jnp.float32)]
               + [jax.ShapeDtypeStruct((_NP, _DH2), jnp.float32)] * 4),
)

_tc_fin = pl.pallas_call(
    _tc_fin_body,
    grid=(_NBLK,),
    in_specs=[
        _row_spec(_D), _row_spec(_DH2), _row_spec(_DH2),
        _row_spec(_DH2), _row_spec(_DH2), _row_spec(128),
        _full_spec((_D, _D)), _full_spec((_DH2, _D)), _full_spec((_DH2, _D)),
        _full_spec((1, _D)),
        _full_spec((_D, _OUT_DIM)), _full_spec((1, _OUT_DIM)),
    ],
    out_specs=pl.BlockSpec((1, _OUT_DIM), lambda i: (0, 0)),
    out_shape=jax.ShapeDtypeStruct((1, _OUT_DIM), jnp.float32),
    scratch_shapes=[pltpu.VMEM((8, _D), jnp.float32)],
)


def _sc_hop_body(a_lo, a_hi, src_hbm, dst_hbm, out_lo, out_hi,
                 srcall, ldstall, pkfl, rows0, rows1, tmp, aggsh,
                 gsem0, gsem1, ssem0, ssem1):
    cid = lax.axis_index("c")
    sid = lax.axis_index("s")
    base_node = cid * _HALF
    crow = sid * _NCHUNK

    pltpu.sync_copy(src_hbm.at[pl.ds(crow, _NCHUNK)], srcall)
    pltpu.sync_copy(dst_hbm.at[pl.ds(crow, _NCHUNK)], ldstall)

    iota16 = lax.iota(jnp.int32, 16)
    trash = _HALF + iota16

    def _pref(i, carry):
        pkfl[pl.ds(i * 16, 16)] = trash
        return carry
    lax.fori_loop(0, _PER_SUB // 16 + 1, _pref, 0)

    def _compact(i, off):
        k = i // (_CH // 16)
        j = i % (_CH // 16)
        d = ldstall[k, pl.ds(j * 16, 16)]
        s = srcall[k, pl.ds(j * 16, 16)]
        l = d - base_node
        keep = (l >= 0) & (l < _HALF)
        packed = s * 8192 + jnp.where(keep, l, _HALF)
        keep01 = jnp.where(keep, 1, 0)
        out = trash
        pos = jnp.int32(0)
        for lane in range(16):
            v_l = packed[lane]
            k_l = keep01[lane]
            tgt = jnp.where(k_l == 1, pos, 16)
            out = jnp.where(iota16 == tgt, v_l, out)
            pos = pos + k_l
        pkfl[pl.ds(off, 16)] = out
        return off + pos
    count = lax.fori_loop(0, _PER_SUB // 16, _compact, jnp.int32(0))
    pkfl[pl.ds(count, 16)] = trash

    def _re2d(i, carry):
        k = i // (_CH // 16)
        c = i % (_CH // 16)
        v = pkfl[pl.ds(i * 16, 16)]
        srcall[k, pl.ds(c * 16, 16)] = lax.shift_right_logical(v, 13)
        ldstall[k, pl.ds(c * 16, 16)] = lax.bitwise_and(v, 8191)
        return carry
    lax.fori_loop(0, _PER_SUB // 16, _re2d, 0)

    npairs = (count + 2 * _CH - 1) // (2 * _CH)
    rows = (rows0, rows1)
    gsems = (gsem0, gsem1)
    ssems = (ssem0, ssem1)

    def _gather(a_hbm, k, b):
        return pltpu.async_copy(a_hbm.at[srcall.at[k]], rows[b], gsems[b])

    def _scatter(k, b):
        return pltpu.async_copy(rows[b], aggsh.at[ldstall.at[k]], add=True,
                                sem=ssems[b])

    def _gwait(a_hbm, b):
        pltpu.make_async_copy(a_hbm.at[srcall.at[0]], rows[b],
                              gsems[b]).wait()

    def _swait(b):
        pltpu.make_async_copy(rows[b], aggsh.at[ldstall.at[0]],
                              ssems[b]).wait()

    for a_hbm, out_hbm in ((a_lo, out_lo), (a_hi, out_hi)):
        def _zrow(r, carry):
            for c in range(_DH2 // 16):
                tmp[r, pl.ds(c * 16, 16)] = jnp.zeros((16,), jnp.float32)
            return carry
        lax.fori_loop(0, _OCH, _zrow, 0)
        for k in range(_NOCH):
            pltpu.sync_copy(tmp, aggsh.at[pl.ds(sid * _ROWS_PER_SUB
                                                + k * _OCH, _OCH)])
        plsc.subcore_barrier()

        for b in range(2):
            _gather(a_hbm, b, b)
        for b in range(2):
            _gwait(a_hbm, b)
            _scatter(b, b)

        def _pair(p, carry):
            a = 2 * p
            for b in range(2):
                _swait(b)
                _gather(a_hbm, a + b, b)
            for b in range(2):
                _gwait(a_hbm, b)
                _scatter(a + b, b)
            return carry
        lax.fori_loop(1, npairs, _pair, 0)
        for b in range(2):
            _swait(b)
        plsc.subcore_barrier()

        for k in range(_NOCH):
            r0 = sid * _ROWS_PER_SUB + k * _OCH
            pltpu.sync_copy(aggsh.at[pl.ds(r0, _OCH)], tmp)
            pltpu.sync_copy(tmp, out_hbm.at[pl.ds(base_node + r0, _OCH)])
        plsc.subcore_barrier()


_DEG_SLICE = _HALF // _NSUB


def _sc_deg_body(dst_hbm, out_hbm, dstv, ldstv, onesv, tmp, degsh):
    cid = lax.axis_index("c")
    sid = lax.axis_index("s")
    base_node = cid * _HALF
    ebase = sid * _PER_SUB
    nbase = sid * _DEG_SLICE


    zv = jnp.zeros((16,), jnp.float32)
    e0 = jnp.where(lax.iota(jnp.int32, 16) == 0, 1.0, 0.0).astype(jnp.float32)

    def _fill(r, carry):
        onesv[r, pl.ds(0, 16)] = e0
        for c in range(1, _DW // 16):
            onesv[r, pl.ds(c * 16, 16)] = zv
        return carry
    lax.fori_loop(0, _CH, _fill, 0)

    def _ztmp(r, carry):
        for c in range(_DW // 16):
            tmp[r, pl.ds(c * 16, 16)] = jnp.zeros((16,), jnp.float32)
        return carry
    lax.fori_loop(0, _DEG_SLICE, _ztmp, 0)
    pltpu.sync_copy(tmp, degsh.at[pl.ds(nbase, _DEG_SLICE)])
    plsc.subcore_barrier()

    def _chunk(k, carry):
        off = ebase + k * _CH
        pltpu.sync_copy(dst_hbm.at[pl.ds(off, _CH)], dstv)
        for j in range(_CH // 16):
            d = dstv[pl.ds(j * 16, 16)]
            l = d - base_node
            oob = (l < 0) | (l >= _HALF)
            ldstv[pl.ds(j * 16, 16)] = jnp.where(oob, _HALF, l)
        pltpu.sync_copy(onesv, degsh.at[ldstv], add=True)
        return carry
    lax.fori_loop(0, _NCHUNK, _chunk, 0)
    plsc.subcore_barrier()

    pltpu.sync_copy(degsh.at[pl.ds(nbase, _DEG_SLICE)], tmp)
    pltpu.sync_copy(tmp, out_hbm.at[pl.ds(base_node + nbase, _DEG_SLICE)])


@functools.cache
def _get_sc_kernels():
    mesh = plsc.VectorSubcoreMesh(core_axis_name="c", subcore_axis_name="s")
    hop = functools.partial(
        pl.kernel,
        mesh=mesh,
        out_type=[jax.ShapeDtypeStruct((_NP, _DH2), jnp.float32)] * 2,
        scratch_types=[
            pltpu.VMEM((_NCHUNK, _CH), jnp.int32),
            pltpu.VMEM((_NCHUNK, _CH), jnp.int32),
            pltpu.VMEM((_PER_SUB + 32,), jnp.int32),
            pltpu.VMEM((_CH, _DH2), jnp.float32),
            pltpu.VMEM((_CH, _DH2), jnp.float32),
            pltpu.VMEM((_OCH, _DH2), jnp.float32),
            pltpu.VMEM_SHARED((_HALF + 16, _DH2), jnp.float32),
            pltpu.SemaphoreType.DMA,
            pltpu.SemaphoreType.DMA,
            pltpu.SemaphoreType.DMA,
            pltpu.SemaphoreType.DMA,
        ],
    )(_sc_hop_body)
    deg = functools.partial(
        pl.kernel,
        mesh=mesh,
        out_type=jax.ShapeDtypeStruct((_NP, 128), jnp.float32),
        scratch_types=[
            pltpu.VMEM((_CH,), jnp.int32),
            pltpu.VMEM((_CH,), jnp.int32),
            pltpu.VMEM((_CH, _DW), jnp.float32),
            pltpu.VMEM((_DEG_SLICE, _DW), jnp.float32),
            pltpu.VMEM_SHARED((_HALF + 1, _DW), jnp.float32),
        ],
    )(_sc_deg_body)
    return hop, deg


def kernel(nodes, edges, node_types, proj_W, proj_b, type_emb,
           msg_W, msg_b, upd_W, upd_b, out_W, out_b):
    nodes_p = jnp.pad(nodes, ((0, _NP - _N), (0, 0)))
    oh = (node_types[:, None] == jnp.arange(16, dtype=jnp.int32)
          ).astype(jnp.float32)
    oh_p = jnp.pad(oh, ((0, _NP - _N), (0, 0)))
    te_p = jnp.pad(type_emb, ((0, 16 - _NTYPES), (0, 0)))

    src = jnp.pad(edges[0], (0, _EP - _E))
    dst = jnp.pad(edges[1], (0, _EP - _E), constant_values=_NP)
    src2 = src.reshape(_EP // _CH, _CH)
    dst2 = dst.reshape(_EP // _CH, _CH)

    sc_hop, sc_deg = _get_sc_kernels()
    deg_col = sc_deg(dst)

    h, alo, ahi, blo, bhi = _tc_pre(
        nodes_p, oh_p, proj_W, proj_b[None, :], te_p,
        msg_W[0, :_D], msg_W[0, _D:], msg_b[0][None, :])
    for i in range(_HOPS):
        agglo, agghi = sc_hop(alo, ahi, src2, dst2)
        u2 = upd_W[i, _D:]
        if i < _HOPS - 1:
            h, alo, ahi, blo, bhi = _tc_mid(
                h, agglo, agghi, blo, bhi, deg_col,
                upd_W[i, :_D], u2[:_DH2], u2[_DH2:],
                upd_b[i][None, :], msg_W[i + 1, :_D],
                msg_W[i + 1, _D:], msg_b[i + 1][None, :])
        else:
            out = _tc_fin(h, agglo, agghi, blo, bhi, deg_col,
                          upd_W[i, :_D], u2[:_DH2],
                          u2[_DH2:], upd_b[i][None, :], out_W, out_b[None, :])
    return out.reshape(_OUT_DIM)


# speedup vs baseline: 1.0018x; 1.0018x over previous
import functools

import jax
import jax.numpy as jnp
from jax import lax
from jax.experimental import pallas as pl
from jax.experimental.pallas import tpu as pltpu
from jax.experimental.pallas import tpu_sc as plsc

_N = 10000
_E = 160000
_IN_FEAT = 32
_D = 256
_OUT_DIM = 256
_HOPS = 3
_NTYPES = 10

_NP = 10240
_EP = 163840
_HALF = _NP // 2
_NSUB = 16
_CH = 128
_PER_SUB = _EP // _NSUB
_NCHUNK = _PER_SUB // _CH
_ROWS_PER_SUB = _HALF // _NSUB
_OCH = 64
_NOCH = _ROWS_PER_SUB // _OCH

_DH2 = _D // 2
_DW = 128
_BR = 1280
_NBLK = _NP // _BR


def _emit_ab(h, w1_ref, w2_ref, mb_ref, alo_ref, ahi_ref, blo_ref, bhi_ref):
    a = jnp.dot(h, w1_ref[...], preferred_element_type=jnp.float32)
    b = jnp.dot(h, w2_ref[...], preferred_element_type=jnp.float32) + mb_ref[...]
    alo_ref[...] = a[:, :_DH2]
    ahi_ref[...] = a[:, _DH2:]
    blo_ref[...] = b[:, :_DH2]
    bhi_ref[...] = b[:, _DH2:]


def _tc_pre_body(nodes_ref, oh_ref, pw_ref, pb_ref, te_ref, w1_ref, w2_ref,
                 mb_ref, h_ref, alo_ref, ahi_ref, blo_ref, bhi_ref):
    h = jnp.dot(nodes_ref[...], pw_ref[...],
                preferred_element_type=jnp.float32) + pb_ref[...]
    h = h + jnp.dot(oh_ref[...], te_ref[...],
                    preferred_element_type=jnp.float32)
    h_ref[...] = h
    _emit_ab(h, w1_ref, w2_ref, mb_ref, alo_ref, ahi_ref, blo_ref, bhi_ref)


def _update(h_ref, agglo_ref, agghi_ref, blo_ref, bhi_ref, deg_ref,
            u1_ref, u2lo_ref, u2hi_ref, ub_ref):
    deg = deg_ref[:, 0:1]
    lo = agglo_ref[...] + deg * blo_ref[...]
    hi = agghi_ref[...] + deg * bhi_ref[...]
    hn = (jnp.dot(h_ref[...], u1_ref[...], preferred_element_type=jnp.float32)
          + jnp.dot(lo, u2lo_ref[...], preferred_element_type=jnp.float32)
          + jnp.dot(hi, u2hi_ref[...], preferred_element_type=jnp.float32)
          + ub_ref[...])
    return jnp.maximum(hn, 0.0)


def _tc_mid_body(h_ref, agglo_ref, agghi_ref, blo_ref, bhi_ref, deg_ref,
                 u1_ref, u2lo_ref, u2hi_ref,
                 ub_ref, w1_ref, w2_ref, mb_ref, hn_ref,
                 alo_ref, ahi_ref, nblo_ref, nbhi_ref):
    hn = _update(h_ref, agglo_ref, agghi_ref, blo_ref, bhi_ref, deg_ref,
                 u1_ref, u2lo_ref, u2hi_ref, ub_ref)
    hn_ref[...] = hn
    _emit_ab(hn, w1_ref, w2_ref, mb_ref, alo_ref, ahi_ref, nblo_ref, nbhi_ref)


def _tc_fin_body(h_ref, agglo_ref, agghi_ref, blo_ref, bhi_ref, deg_ref,
                 u1_ref, u2lo_ref, u2hi_ref,
                 ub_ref, ow_ref, ob_ref, out_ref, acc_ref):
    i = pl.program_id(0)
    hn = _update(h_ref, agglo_ref, agghi_ref, blo_ref, bhi_ref, deg_ref,
                 u1_ref, u2lo_ref, u2hi_ref, ub_ref)
    row = i * _BR + lax.broadcasted_iota(jnp.int32, (_BR, _D), 0)
    hm = jnp.where(row < _N, hn, -1e30)
    m = jnp.max(hm, axis=0, keepdims=True)

    @pl.when(i == 0)
    def _():
        acc_ref[...] = jnp.full((8, _D), -1e30, jnp.float32)

    acc_ref[0:1, :] = jnp.maximum(acc_ref[0:1, :], m)

    @pl.when(i == _NBLK - 1)
    def _():
        g = acc_ref[0:1, :]
        out_ref[...] = jnp.dot(g, ow_ref[...],
                               preferred_element_type=jnp.float32) + ob_ref[...]


def _row_spec(last):
    return pl.BlockSpec((_BR, last), lambda i: (i, 0))


def _full_spec(shape):
    return pl.BlockSpec(shape, lambda i: (0, 0))


_tc_pre = pl.pallas_call(
    _tc_pre_body,
    grid=(_NBLK,),
    in_specs=[
        _row_spec(_IN_FEAT), _row_spec(16),
        _full_spec((_IN_FEAT, _D)), _full_spec((1, _D)),
        _full_spec((16, _D)), _full_spec((_D, _D)), _full_spec((_D, _D)),
        _full_spec((1, _D)),
    ],
    out_specs=[_row_spec(_D)] + [_row_spec(_DH2)] * 4,
    out_shape=([jax.ShapeDtypeStruct((_NP, _D), jnp.float32)]
               + [jax.ShapeDtypeStruct((_NP, _DH2), jnp.float32)] * 4),
)

_tc_mid = pl.pallas_call(
    _tc_mid_body,
    grid=(_NBLK,),
    in_specs=[
        _row_spec(_D), _row_spec(_DH2), _row_spec(_DH2),
        _row_spec(_DH2), _row_spec(_DH2), _row_spec(128),
        _full_spec((_D, _D)), _full_spec((_DH2, _D)), _full_spec((_DH2, _D)),
        _full_spec((1, _D)),
        _full_spec((_D, _D)), _full_spec((_D, _D)), _full_spec((1, _D)),
    ],
    out_specs=[_row_spec(_D)] + [_row_spec(_DH2)] * 4,
    out_shape=([jax.ShapeDtypeStruct((_NP, _D), jnp.float32)]
               + [jax.ShapeDtypeStruct((_NP, _DH2), jnp.float32)] * 4),
)

_tc_fin = pl.pallas_call(
    _tc_fin_body,
    grid=(_NBLK,),
    in_specs=[
        _row_spec(_D), _row_spec(_DH2), _row_spec(_DH2),
        _row_spec(_DH2), _row_spec(_DH2), _row_spec(128),
        _full_spec((_D, _D)), _full_spec((_DH2, _D)), _full_spec((_DH2, _D)),
        _full_spec((1, _D)),
        _full_spec((_D, _OUT_DIM)), _full_spec((1, _OUT_DIM)),
    ],
    out_specs=pl.BlockSpec((1, _OUT_DIM), lambda i: (0, 0)),
    out_shape=jax.ShapeDtypeStruct((1, _OUT_DIM), jnp.float32),
    scratch_shapes=[pltpu.VMEM((8, _D), jnp.float32)],
)


def _sc_hop_body(a_lo, a_hi, src_hbm, dst_hbm, out_lo, out_hi,
                 srcall, ldstall, pkfl, rows0, rows1, tmp, aggsh,
                 gsem0, gsem1, ssem0, ssem1):
    cid = lax.axis_index("c")
    sid = lax.axis_index("s")
    base_node = cid * _HALF
    crow = sid * _NCHUNK

    pltpu.sync_copy(src_hbm.at[pl.ds(crow, _NCHUNK)], srcall)
    pltpu.sync_copy(dst_hbm.at[pl.ds(crow, _NCHUNK)], ldstall)

    iota16 = lax.iota(jnp.int32, 16)
    trash = _HALF + iota16

    def _pref(i, carry):
        pkfl[pl.ds(i * 16, 16)] = trash
        return carry
    lax.fori_loop(0, _PER_SUB // 16 + 1, _pref, 0)

    def _compact(i, off):
        k = i // (_CH // 16)
        j = i % (_CH // 16)
        d = ldstall[k, pl.ds(j * 16, 16)]
        s = srcall[k, pl.ds(j * 16, 16)]
        l = d - base_node
        keep = (l >= 0) & (l < _HALF)
        packed = s * 8192 + jnp.where(keep, l, _HALF)
        keep01 = jnp.where(keep, 1, 0)
        out = trash
        pos = jnp.int32(0)
        for lane in range(16):
            v_l = packed[lane]
            k_l = keep01[lane]
            tgt = jnp.where(k_l == 1, pos, 16)
            out = jnp.where(iota16 == tgt, v_l, out)
            pos = pos + k_l
        pkfl[pl.ds(off, 16)] = out
        return off + pos
    count = lax.fori_loop(0, _PER_SUB // 16, _compact, jnp.int32(0))
    pkfl[pl.ds(count, 16)] = trash

    def _re2d(i, carry):
        k = i // (_CH // 16)
        c = i % (_CH // 16)
        v = pkfl[pl.ds(i * 16, 16)]
        srcall[k, pl.ds(c * 16, 16)] = lax.shift_right_logical(v, 13)
        ldstall[k, pl.ds(c * 16, 16)] = lax.bitwise_and(v, 8191)
        return carry
    lax.fori_loop(0, _PER_SUB // 16, _re2d, 0)

    npairs = (count + 2 * _CH - 1) // (2 * _CH)
    rows = (rows0, rows1)
    gsems = (gsem0, gsem1)
    ssems = (ssem0, ssem1)

    def _gather(a_hbm, k, b):
        return pltpu.async_copy(a_hbm.at[srcall.at[k]], rows[b], gsems[b])

    def _scatter(k, b):
        return pltpu.async_copy(rows[b], aggsh.at[ldstall.at[k]], add=True,
                                sem=ssems[b])

    def _gwait(a_hbm, b):
        pltpu.make_async_copy(a_hbm.at[srcall.at[0]], rows[b],
                              gsems[b]).wait()

    def _swait(b):
        pltpu.make_async_copy(rows[b], aggsh.at[ldstall.at[0]],
                              ssems[b]).wait()

    def _zrow(r, carry):
        for c in range(_DH2 // 16):
            tmp[r, pl.ds(c * 16, 16)] = jnp.zeros((16,), jnp.float32)
        return carry
    lax.fori_loop(0, _OCH, _zrow, 0)

    for a_hbm, out_hbm in ((a_lo, out_lo), (a_hi, out_hi)):
        for k in range(_NOCH):
            pltpu.async_copy(tmp, aggsh.at[pl.ds(sid * _ROWS_PER_SUB
                                                 + k * _OCH, _OCH)], gsem0)
        for k in range(_NOCH):
            pltpu.make_async_copy(tmp, aggsh.at[pl.ds(0, _OCH)],
                                  gsem0).wait()
        plsc.subcore_barrier()

        for b in range(2):
            _gather(a_hbm, b, b)
        for b in range(2):
            _gwait(a_hbm, b)
            _scatter(b, b)

        def _pair(p, carry):
            a = 2 * p
            for b in range(2):
                _swait(b)
                _gather(a_hbm, a + b, b)
            for b in range(2):
                _gwait(a_hbm, b)
                _scatter(a + b, b)
            return carry
        lax.fori_loop(1, npairs, _pair, 0)
        for b in range(2):
            _swait(b)
        plsc.subcore_barrier()

        for k in range(_NOCH):
            r0 = sid * _ROWS_PER_SUB + k * _OCH
            pltpu.async_copy(aggsh.at[pl.ds(r0, _OCH)],
                             out_hbm.at[pl.ds(base_node + r0, _OCH)], gsem0)
        for k in range(_NOCH):
            pltpu.make_async_copy(aggsh.at[pl.ds(0, _OCH)],
                                  out_hbm.at[pl.ds(0, _OCH)], gsem0).wait()
        plsc.subcore_barrier()


_DEG_SLICE = _HALF // _NSUB


def _sc_deg_body(dst_hbm, out_hbm, dstv, ldstv, onesv, tmp, degsh):
    cid = lax.axis_index("c")
    sid = lax.axis_index("s")
    base_node = cid * _HALF
    ebase = sid * _PER_SUB
    nbase = sid * _DEG_SLICE


    zv = jnp.zeros((16,), jnp.float32)
    e0 = jnp.where(lax.iota(jnp.int32, 16) == 0, 1.0, 0.0).astype(jnp.float32)

    def _fill(r, carry):
        onesv[r, pl.ds(0, 16)] = e0
        for c in range(1, _DW // 16):
            onesv[r, pl.ds(c * 16, 16)] = zv
        return carry
    lax.fori_loop(0, _CH, _fill, 0)

    def _ztmp(r, carry):
        for c in range(_DW // 16):
            tmp[r, pl.ds(c * 16, 16)] = jnp.zeros((16,), jnp.float32)
        return carry
    lax.fori_loop(0, _DEG_SLICE, _ztmp, 0)
    pltpu.sync_copy(tmp, degsh.at[pl.ds(nbase, _DEG_SLICE)])
    plsc.subcore_barrier()

    def _chunk(k, carry):
        off = ebase + k * _CH
        pltpu.sync_copy(dst_hbm.at[pl.ds(off, _CH)], dstv)
        for j in range(_CH // 16):
            d = dstv[pl.ds(j * 16, 16)]
            l = d - base_node
            oob = (l < 0) | (l >= _HALF)
            ldstv[pl.ds(j * 16, 16)] = jnp.where(oob, _HALF, l)
        pltpu.sync_copy(onesv, degsh.at[ldstv], add=True)
        return carry
    lax.fori_loop(0, _NCHUNK, _chunk, 0)
    plsc.subcore_barrier()

    pltpu.sync_copy(degsh.at[pl.ds(nbase, _DEG_SLICE)], tmp)
    pltpu.sync_copy(tmp, out_hbm.at[pl.ds(base_node + nbase, _DEG_SLICE)])


@functools.cache
def _get_sc_kernels():
    mesh = plsc.VectorSubcoreMesh(core_axis_name="c", subcore_axis_name="s")
    hop = functools.partial(
        pl.kernel,
        mesh=mesh,
        out_type=[jax.ShapeDtypeStruct((_NP, _DH2), jnp.float32)] * 2,
        scratch_types=[
            pltpu.VMEM((_NCHUNK, _CH), jnp.int32),
            pltpu.VMEM((_NCHUNK, _CH), jnp.int32),
            pltpu.VMEM((_PER_SUB + 32,), jnp.int32),
            pltpu.VMEM((_CH, _DH2), jnp.float32),
            pltpu.VMEM((_CH, _DH2), jnp.float32),
            pltpu.VMEM((_OCH, _DH2), jnp.float32),
            pltpu.VMEM_SHARED((_HALF + 16, _DH2), jnp.float32),
            pltpu.SemaphoreType.DMA,
            pltpu.SemaphoreType.DMA,
            pltpu.SemaphoreType.DMA,
            pltpu.SemaphoreType.DMA,
        ],
    )(_sc_hop_body)
    deg = functools.partial(
        pl.kernel,
        mesh=mesh,
        out_type=jax.ShapeDtypeStruct((_NP, 128), jnp.float32),
        scratch_types=[
            pltpu.VMEM((_CH,), jnp.int32),
            pltpu.VMEM((_CH,), jnp.int32),
            pltpu.VMEM((_CH, _DW), jnp.float32),
            pltpu.VMEM((_DEG_SLICE, _DW), jnp.float32),
            pltpu.VMEM_SHARED((_HALF + 1, _DW), jnp.float32),
        ],
    )(_sc_deg_body)
    return hop, deg


def kernel(nodes, edges, node_types, proj_W, proj_b, type_emb,
           msg_W, msg_b, upd_W, upd_b, out_W, out_b):
    nodes_p = jnp.pad(nodes, ((0, _NP - _N), (0, 0)))
    oh = (node_types[:, None] == jnp.arange(16, dtype=jnp.int32)
          ).astype(jnp.float32)
    oh_p = jnp.pad(oh, ((0, _NP - _N), (0, 0)))
    te_p = jnp.pad(type_emb, ((0, 16 - _NTYPES), (0, 0)))

    src = jnp.pad(edges[0], (0, _EP - _E))
    dst = jnp.pad(edges[1], (0, _EP - _E), constant_values=_NP)
    src2 = src.reshape(_EP // _CH, _CH)
    dst2 = dst.reshape(_EP // _CH, _CH)

    sc_hop, sc_deg = _get_sc_kernels()
    deg_col = sc_deg(dst)

    h, alo, ahi, blo, bhi = _tc_pre(
        nodes_p, oh_p, proj_W, proj_b[None, :], te_p,
        msg_W[0, :_D], msg_W[0, _D:], msg_b[0][None, :])
    for i in range(_HOPS):
        agglo, agghi = sc_hop(alo, ahi, src2, dst2)
        u2 = upd_W[i, _D:]
        if i < _HOPS - 1:
            h, alo, ahi, blo, bhi = _tc_mid(
                h, agglo, agghi, blo, bhi, deg_col,
                upd_W[i, :_D], u2[:_DH2], u2[_DH2:],
                upd_b[i][None, :], msg_W[i + 1, :_D],
                msg_W[i + 1, _D:], msg_b[i + 1][None, :])
        else:
            out = _tc_fin(h, agglo, agghi, blo, bhi, deg_col,
                          upd_W[i, :_D], u2[:_DH2],
                          u2[_DH2:], upd_b[i][None, :], out_W, out_b[None, :])
    return out.reshape(_OUT_DIM)
